# baseline (speedup 1.0000x reference)
import jax
import jax.numpy as jnp
from jax.experimental import pallas as pl
from jax.experimental.pallas import tpu as pltpu

_BM = 640


def _gcn2_block(scal_ref, adj_ref, x_ref, h0_ref, w_ref, out_ref):
    alpha = scal_ref[0]
    theta = scal_ref[1]
    hi = jnp.dot(adj_ref[...], x_ref[...], preferred_element_type=jnp.float32)
    support = (1.0 - alpha) * hi + alpha * h0_ref[...]
    out_ref[...] = (
        theta * jnp.dot(support, w_ref[...], preferred_element_type=jnp.float32)
        + (1.0 - theta) * support
    )


def kernel(input, adj, adj_high, h0, lamda, alpha, l, weight_low):
    n, d = input.shape
    theta = jnp.log(lamda / l + 1.0)
    scal = jnp.stack([alpha.astype(jnp.float32), theta.astype(jnp.float32)])
    return pl.pallas_call(
        _gcn2_block,
        grid=(pl.cdiv(n, _BM),),
        in_specs=[
            pl.BlockSpec(memory_space=pltpu.SMEM),
            pl.BlockSpec((_BM, n), lambda i: (i, 0)),
            pl.BlockSpec((n, d), lambda i: (0, 0)),
            pl.BlockSpec((_BM, d), lambda i: (i, 0)),
            pl.BlockSpec((d, d), lambda i: (0, 0)),
        ],
        out_specs=pl.BlockSpec((_BM, d), lambda i: (i, 0)),
        out_shape=jax.ShapeDtypeStruct((n, d), jnp.float32),
        compiler_params=pltpu.CompilerParams(
            dimension_semantics=("arbitrary",),
        ),
    )(scal, adj, input, h0, weight_low)


# speedup vs baseline: 1.0140x; 1.0140x over previous
import jax
import jax.numpy as jnp
from jax.experimental import pallas as pl
from jax.experimental.pallas import tpu as pltpu

_BM = 480


def _gcn2_block(scal_ref, adj_ref, x_ref, h0_ref, w_ref, out_ref):
    alpha = scal_ref[0]
    theta = scal_ref[1]
    hi = jnp.dot(adj_ref[...], x_ref[...], preferred_element_type=jnp.float32)
    support = (1.0 - alpha) * hi + alpha * h0_ref[...]
    out_ref[...] = (
        theta * jnp.dot(support, w_ref[...], preferred_element_type=jnp.float32)
        + (1.0 - theta) * support
    )


def kernel(input, adj, adj_high, h0, lamda, alpha, l, weight_low):
    n, d = input.shape
    theta = jnp.log(lamda / l + 1.0)
    scal = jnp.stack([alpha.astype(jnp.float32), theta.astype(jnp.float32)])
    return pl.pallas_call(
        _gcn2_block,
        grid=(pl.cdiv(n, _BM),),
        in_specs=[
            pl.BlockSpec(memory_space=pltpu.SMEM),
            pl.BlockSpec((_BM, n), lambda i: (i, 0)),
            pl.BlockSpec((n, d), lambda i: (0, 0)),
            pl.BlockSpec((_BM, d), lambda i: (i, 0)),
            pl.BlockSpec((d, d), lambda i: (0, 0)),
        ],
        out_specs=pl.BlockSpec((_BM, d), lambda i: (i, 0)),
        out_shape=jax.ShapeDtypeStruct((n, d), jnp.float32),
        compiler_params=pltpu.CompilerParams(
            dimension_semantics=("arbitrary",),
        ),
    )(scal, adj, input, h0, weight_low)
